# baseline (speedup 1.0000x reference)
import functools

import jax
import jax.numpy as jnp
from jax import lax
from jax.experimental import pallas as pl
from jax.experimental.pallas import tpu as pltpu
from jax.experimental.pallas import tpu_sc as plsc

N = 10000
E = 320000
D = 128
NC = 2
NS = 16
K = 64
U = 6
CHUNKS = 318
CPAD = CHUNKS + U
RPT = 624
BM = 1000


def _matmul_body(x_ref, w_ref, o_ref):
    o_ref[0] = jnp.dot(x_ref[...], w_ref[0],
                       preferred_element_type=jnp.float32)


_matmul = pl.pallas_call(
    _matmul_body,
    grid=(2, N // BM),
    in_specs=[
        pl.BlockSpec((BM, D), lambda g, i: (i, 0)),
        pl.BlockSpec((1, D, D), lambda g, i: (g, 0, 0)),
    ],
    out_specs=pl.BlockSpec((1, BM, D), lambda g, i: (g, i, 0)),
    out_shape=jax.ShapeDtypeStruct((2, N, D), jnp.float32),
)


_sc_mesh = plsc.VectorSubcoreMesh(core_axis_name="c", subcore_axis_name="s")


@functools.partial(
    pl.kernel,
    out_type=jax.ShapeDtypeStruct((NC, N, D), jnp.float32),
    mesh=_sc_mesh,
    scratch_types=[
        pltpu.VMEM((2, U, 2, K), jnp.int32),
        pltpu.VMEM((2, U, 1, K), jnp.float32),
        pltpu.VMEM((K, D), jnp.float32),
        pltpu.VMEM((K, D), jnp.float32),
        pltpu.VMEM((K, D), jnp.float32),
        pltpu.VMEM((K, D), jnp.float32),
        pltpu.VMEM((K, D), jnp.float32),
        pltpu.VMEM_SHARED((N, D), jnp.float32),
        pltpu.SemaphoreType.DMA,
        pltpu.SemaphoreType.DMA,
        pltpu.SemaphoreType.DMA,
        pltpu.SemaphoreType.DMA,
        pltpu.SemaphoreType.DMA,
        pltpu.SemaphoreType.DMA,
    ],
)
def _spmm_kernel(sup_hbm, idx_hbm, vals_hbm, out_hbm,
                 ibuf, vbuf, g0, g1, g2, s0, s1, acc,
                 gsem0, gsem1, gsem2, ssem0, ssem1, bsem):
    c = lax.axis_index("c")
    s = lax.axis_index("s")
    gbufs = (g0, g1, g2)
    sbufs = (s0, s1)
    gsems = (gsem0, gsem1, gsem2)
    ssems = (ssem0, ssem1)

    def bigload(b, pb):
        pltpu.async_copy(idx_hbm.at[c, s, pl.ds(b * U, U)], ibuf.at[pb], bsem)
        pltpu.async_copy(vals_hbm.at[c, s, pl.ds(b * U, U)], vbuf.at[pb],
                         bsem)

    def bigload_wait(pb):
        pltpu.make_async_copy(idx_hbm.at[c, s, pl.ds(0, U)], ibuf.at[pb],
                              bsem).wait()
        pltpu.make_async_copy(vals_hbm.at[c, s, pl.ds(0, U)], vbuf.at[pb],
                              bsem).wait()

    def gather(pb, slot, p):
        pltpu.async_copy(sup_hbm.at[c].at[ibuf.at[pb, slot, 1]], gbufs[p],
                         gsems[p])

    def gather_wait(p):
        pltpu.make_async_copy(sup_hbm.at[c, pl.ds(0, K), :], gbufs[p],
                              gsems[p]).wait()

    def scat(pb, slot, p):
        pltpu.async_copy(sbufs[p], acc.at[ibuf.at[pb, slot, 0]], ssems[p],
                         add=True)

    def scat_wait(p):
        pltpu.make_async_copy(sbufs[p], acc.at[pl.ds(0, K), :],
                              ssems[p]).wait()

    def scale(pb, slot, gp, sp):
        g = gbufs[gp]
        sb = sbufs[sp]

        @plsc.parallel_loop(0, K // 16)
        def _sbody(gr):
            vv = vbuf[pb, slot, 0, pl.ds(gr * 16, 16)]
            for l in range(16):
                v = vv[l]
                i = gr * 16 + l
                for jj in range(D // 16):
                    sl = pl.ds(jj * 16, 16)
                    sb[i, sl] = g[i, sl] * v

    def step(b, j, pb, skip_scat_wait=False):
        gp = j % 3
        sp = j % 2
        if j == 3:
            bigload_wait(1 - pb)
        if j < 4:
            gather(pb, j + 2, (j + 2) % 3)
        else:
            gather(1 - pb, j - 4, (j + 2) % 3)
        gather_wait(gp)
        if not skip_scat_wait:
            scat_wait(sp)
        if j == 1:
            bigload(b + 1, 1 - pb)
        scale(pb, j, gp, sp)
        scat(pb, j, sp)

    zero16 = jnp.zeros((16,), jnp.float32)

    def zbody(i, _):
        for j in range(D // 16):
            s0[i, pl.ds(j * 16, 16)] = zero16
        return 0

    lax.fori_loop(0, K, zbody, 0)
    base = s * RPT
    for t in range(RPT // K):
        pltpu.sync_copy(s0, acc.at[pl.ds(base + t * K, K), :])
    if RPT % K:
        pltpu.sync_copy(s0.at[pl.ds(0, RPT % K), :],
                        acc.at[pl.ds(base + (RPT // K) * K, RPT % K), :])

    @pl.when(s == NS - 1)
    def _zero_tail():
        pltpu.sync_copy(s0.at[pl.ds(0, N - NS * RPT), :],
                        acc.at[pl.ds(NS * RPT, N - NS * RPT), :])

    plsc.subcore_barrier()

    bigload(0, 0)
    bigload_wait(0)
    gather(0, 0, 0)
    gather(0, 1, 1)

    step(0, 0, 0, skip_scat_wait=True)
    step(0, 1, 0, skip_scat_wait=True)
    step(0, 2, 0)
    step(0, 3, 0)
    step(0, 4, 0)
    step(0, 5, 0)

    def block_body(b, _):
        pb = b % 2
        for j in range(U):
            step(b, j, pb)
        return 0

    lax.fori_loop(1, CHUNKS // U, block_body, 0)

    scat_wait(CHUNKS % 2)
    scat_wait((CHUNKS + 1) % 2)
    gather_wait(CHUNKS % 3)
    gather_wait((CHUNKS + 1) % 3)

    plsc.subcore_barrier()
    pltpu.sync_copy(acc.at[pl.ds(base, RPT), :],
                    out_hbm.at[c, pl.ds(base, RPT), :])

    @pl.when(s == NS - 1)
    def _drain_tail():
        pltpu.sync_copy(acc.at[pl.ds(NS * RPT, N - NS * RPT), :],
                        out_hbm.at[c, pl.ds(NS * RPT, N - NS * RPT), :])


def _combine_body(p_ref, b_ref, o_ref):
    o_ref[...] = p_ref[0] + p_ref[1] + b_ref[...]


_combine = pl.pallas_call(
    _combine_body,
    grid=(N // BM,),
    in_specs=[
        pl.BlockSpec((2, BM, D), lambda i: (0, i, 0)),
        pl.BlockSpec((1, D), lambda i: (0, 0)),
    ],
    out_specs=pl.BlockSpec((BM, D), lambda i: (i, 0)),
    out_shape=jax.ShapeDtypeStruct((N, D), jnp.float32),
)


def _pad_rs(a):
    per_tile = E // NS
    a = a.reshape(NS, per_tile)
    a = jnp.pad(a, ((0, 0), (0, CPAD * K - per_tile)))
    return a.reshape(NS, CPAD, K)


def _prep_idx(rows, cols):
    return jnp.stack([_pad_rs(rows.astype(jnp.int32)),
                      _pad_rs(cols.astype(jnp.int32))], axis=2)


def _prep_val(vals):
    return _pad_rs(vals.astype(jnp.float32))[:, :, None, :]


def kernel(input, weight_1, weight_2, bias,
           adj0_rows, adj0_cols, adj0_vals,
           adj1_rows, adj1_cols, adj1_vals):
    w = jnp.stack([weight_1, weight_2])
    sup = _matmul(input, w)
    idx = jnp.stack([_prep_idx(adj0_rows, adj0_cols),
                     _prep_idx(adj1_rows, adj1_cols)])
    vals = jnp.stack([_prep_val(adj0_vals), _prep_val(adj1_vals)])
    partial = _spmm_kernel(sup, idx, vals)
    return _combine(partial, bias.reshape(1, D))


# speedup vs baseline: 1.2075x; 1.2075x over previous
import functools

import jax
import jax.numpy as jnp
from jax import lax
from jax.experimental import pallas as pl
from jax.experimental.pallas import tpu as pltpu
from jax.experimental.pallas import tpu_sc as plsc

N = 10000
E = 320000
D = 128
NC = 2
NS = 16
K = 80
U = 6
CHUNKS = 252
CPAD = CHUNKS + U
RPT = 624
BM = 1000


def _matmul_body(x_ref, w_ref, o_ref):
    o_ref[0] = jnp.dot(x_ref[...], w_ref[0],
                       preferred_element_type=jnp.float32)


_matmul = pl.pallas_call(
    _matmul_body,
    grid=(2, N // BM),
    in_specs=[
        pl.BlockSpec((BM, D), lambda g, i: (i, 0)),
        pl.BlockSpec((1, D, D), lambda g, i: (g, 0, 0)),
    ],
    out_specs=pl.BlockSpec((1, BM, D), lambda g, i: (g, i, 0)),
    out_shape=jax.ShapeDtypeStruct((2, N, D), jnp.float32),
)


_sc_mesh = plsc.VectorSubcoreMesh(core_axis_name="c", subcore_axis_name="s")


@functools.partial(
    pl.kernel,
    out_type=jax.ShapeDtypeStruct((NC, N, D), jnp.float32),
    mesh=_sc_mesh,
    scratch_types=[
        pltpu.VMEM((2, U, 2, K), jnp.int32),
        pltpu.VMEM((2, U, 1, K), jnp.float32),
        pltpu.VMEM((K, D), jnp.float32),
        pltpu.VMEM((K, D), jnp.float32),
        pltpu.VMEM((K, D), jnp.float32),
        pltpu.VMEM((K, D), jnp.float32),
        pltpu.VMEM_SHARED((N, D), jnp.float32),
        pltpu.SemaphoreType.DMA,
        pltpu.SemaphoreType.DMA,
        pltpu.SemaphoreType.DMA,
        pltpu.SemaphoreType.DMA,
        pltpu.SemaphoreType.DMA,
    ],
)
def _spmm_kernel(sup_hbm, idx_hbm, vals_hbm, out_hbm,
                 ibuf, vbuf, g0, g1, s0, s1, acc,
                 gsem0, gsem1, ssem0, ssem1, bsem):
    c = lax.axis_index("c")
    s = lax.axis_index("s")
    gbufs = (g0, g1)
    sbufs = (s0, s1)
    gsems = (gsem0, gsem1)
    ssems = (ssem0, ssem1)

    def bigload(b, pb):
        pltpu.async_copy(idx_hbm.at[c, s, pl.ds(b * U, U)], ibuf.at[pb], bsem)
        pltpu.async_copy(vals_hbm.at[c, s, pl.ds(b * U, U)], vbuf.at[pb],
                         bsem)

    def bigload_wait(pb):
        pltpu.make_async_copy(idx_hbm.at[c, s, pl.ds(0, U)], ibuf.at[pb],
                              bsem).wait()
        pltpu.make_async_copy(vals_hbm.at[c, s, pl.ds(0, U)], vbuf.at[pb],
                              bsem).wait()

    def gather(pb, slot, p):
        pltpu.async_copy(sup_hbm.at[c].at[ibuf.at[pb, slot, 1]], gbufs[p],
                         gsems[p])

    def gather_wait(p):
        pltpu.make_async_copy(sup_hbm.at[c, pl.ds(0, K), :], gbufs[p],
                              gsems[p]).wait()

    def scat(pb, slot, p):
        pltpu.async_copy(sbufs[p], acc.at[ibuf.at[pb, slot, 0]], ssems[p],
                         add=True)

    def scat_wait(p):
        pltpu.make_async_copy(sbufs[p], acc.at[pl.ds(0, K), :],
                              ssems[p]).wait()

    def scale(pb, slot, gp, sp):
        g = gbufs[gp]
        sb = sbufs[sp]

        @plsc.parallel_loop(0, K // 16)
        def _sbody(gr):
            vv = vbuf[pb, slot, 0, pl.ds(gr * 16, 16)]
            for l in range(16):
                v = vv[l]
                i = gr * 16 + l
                for jj in range(D // 16):
                    sl = pl.ds(jj * 16, 16)
                    sb[i, sl] = g[i, sl] * v

    def step(b, j, pb, skip_scat_wait=False):
        p = j % 2
        o = 1 - p
        if j == 4:
            bigload_wait(1 - pb)
        if j < 5:
            gather(pb, j + 1, o)
        else:
            gather(1 - pb, 0, o)
        gather_wait(p)
        if not skip_scat_wait:
            scat_wait(p)
        if j == 1:
            bigload(b + 1, 1 - pb)
        scale(pb, j, p, p)
        scat(pb, j, p)

    zero16 = jnp.zeros((16,), jnp.float32)

    def zbody(i, _):
        for j in range(D // 16):
            s0[i, pl.ds(j * 16, 16)] = zero16
        return 0

    lax.fori_loop(0, K, zbody, 0)
    base = s * RPT
    for t in range(RPT // K):
        pltpu.sync_copy(s0, acc.at[pl.ds(base + t * K, K), :])
    if RPT % K:
        pltpu.sync_copy(s0.at[pl.ds(0, RPT % K), :],
                        acc.at[pl.ds(base + (RPT // K) * K, RPT % K), :])

    @pl.when(s == NS - 1)
    def _zero_tail():
        pltpu.sync_copy(s0.at[pl.ds(0, N - NS * RPT), :],
                        acc.at[pl.ds(NS * RPT, N - NS * RPT), :])

    plsc.subcore_barrier()

    bigload(0, 0)
    bigload_wait(0)
    gather(0, 0, 0)

    step(0, 0, 0, skip_scat_wait=True)
    step(0, 1, 0, skip_scat_wait=True)
    step(0, 2, 0)
    step(0, 3, 0)
    step(0, 4, 0)
    step(0, 5, 0)

    def block_body(b, _):
        pb = b % 2
        for j in range(U):
            step(b, j, pb)
        return 0

    lax.fori_loop(1, CHUNKS // U, block_body, 0)

    scat_wait(CHUNKS % 2)
    scat_wait((CHUNKS + 1) % 2)
    gather_wait(CHUNKS % 2)

    plsc.subcore_barrier()
    pltpu.sync_copy(acc.at[pl.ds(base, RPT), :],
                    out_hbm.at[c, pl.ds(base, RPT), :])

    @pl.when(s == NS - 1)
    def _drain_tail():
        pltpu.sync_copy(acc.at[pl.ds(NS * RPT, N - NS * RPT), :],
                        out_hbm.at[c, pl.ds(NS * RPT, N - NS * RPT), :])


def _combine_body(p_ref, b_ref, o_ref):
    o_ref[...] = p_ref[0] + p_ref[1] + b_ref[...]


_combine = pl.pallas_call(
    _combine_body,
    grid=(N // BM,),
    in_specs=[
        pl.BlockSpec((2, BM, D), lambda i: (0, i, 0)),
        pl.BlockSpec((1, D), lambda i: (0, 0)),
    ],
    out_specs=pl.BlockSpec((BM, D), lambda i: (i, 0)),
    out_shape=jax.ShapeDtypeStruct((N, D), jnp.float32),
)


def _pad_rs(a):
    per_tile = E // NS
    a = a.reshape(NS, per_tile)
    a = jnp.pad(a, ((0, 0), (0, CPAD * K - per_tile)))
    return a.reshape(NS, CPAD, K)


def _prep_idx(rows, cols):
    return jnp.stack([_pad_rs(rows.astype(jnp.int32)),
                      _pad_rs(cols.astype(jnp.int32))], axis=2)


def _prep_val(vals):
    return _pad_rs(vals.astype(jnp.float32))[:, :, None, :]


def kernel(input, weight_1, weight_2, bias,
           adj0_rows, adj0_cols, adj0_vals,
           adj1_rows, adj1_cols, adj1_vals):
    w = jnp.stack([weight_1, weight_2])
    sup = _matmul(input, w)
    idx = jnp.stack([_prep_idx(adj0_rows, adj0_cols),
                     _prep_idx(adj1_rows, adj1_cols)])
    vals = jnp.stack([_prep_val(adj0_vals), _prep_val(adj1_vals)])
    partial = _spmm_kernel(sup, idx, vals)
    return _combine(partial, bias.reshape(1, D))
